# Initial kernel scaffold; baseline (speedup 1.0000x reference)
import jax
import jax.numpy as jnp
from jax.experimental import pallas as pl


def kernel(x, table):
    raise NotImplementedError("write your pallas kernel here")


# speedup vs baseline: 4.9513x; 4.9513x over previous
import functools
import math

import jax
import jax.numpy as jnp
from jax import lax
from jax.experimental import pallas as pl
from jax.experimental.pallas import tpu as pltpu
from jax.experimental.pallas import tpu_sc as plsc

D_MODEL = 128
SCALE = math.sqrt(float(D_MODEL))

NUM_CORES = 2
NUM_SUBCORES = 16
NUM_WORKERS = NUM_CORES * NUM_SUBCORES
LANES = 16

B_TOTAL = 1024 * 200
B_PER_W = B_TOTAL // NUM_WORKERS
CHUNK = 128
NCHUNK = B_PER_W // CHUNK
IDX_COLS = 128
IDX_ROWS_PER_W = B_PER_W // IDX_COLS


def _emb_kernel(idx_hbm, table_hbm, out_hbm, idx_v, buf, sem):
    wid = lax.axis_index("s") * NUM_CORES + lax.axis_index("c")

    pltpu.sync_copy(idx_hbm.at[wid], idx_v)

    def chunk_body(g, carry):
        pltpu.async_copy(table_hbm.at[idx_v.at[g]], buf, sem).wait()

        def row_body(i, c):
            for j in range(D_MODEL // LANES):
                sl = pl.ds(j * LANES, LANES)
                buf[i, sl] = buf[i, sl] * SCALE
            return c

        lax.fori_loop(0, CHUNK, row_body, 0)

        row0 = (wid * NCHUNK + g) * CHUNK
        pltpu.sync_copy(buf, out_hbm.at[pl.ds(row0, CHUNK)])
        return carry

    lax.fori_loop(0, NCHUNK, chunk_body, 0)


@functools.partial(jax.jit, static_argnames=())
def kernel(x, table):
    idx3d = x.reshape(NUM_WORKERS, IDX_ROWS_PER_W, IDX_COLS)
    mesh = plsc.VectorSubcoreMesh(core_axis_name="c", subcore_axis_name="s")
    out = pl.kernel(
        _emb_kernel,
        mesh=mesh,
        out_type=jax.ShapeDtypeStruct((B_TOTAL, D_MODEL), jnp.float32),
        scratch_types=[
            pltpu.VMEM((IDX_ROWS_PER_W, IDX_COLS), jnp.int32),
            pltpu.VMEM((CHUNK, D_MODEL), jnp.float32),
            pltpu.SemaphoreType.DMA,
        ],
    )(idx3d, table)
    return out.reshape(x.shape[0], x.shape[1], D_MODEL)


# speedup vs baseline: 8.3579x; 1.6880x over previous
import functools
import math

import jax
import jax.numpy as jnp
from jax import lax
from jax.experimental import pallas as pl
from jax.experimental.pallas import tpu as pltpu
from jax.experimental.pallas import tpu_sc as plsc

D_MODEL = 128
SCALE = math.sqrt(float(D_MODEL))

NUM_CORES = 2
NUM_SUBCORES = 16
NUM_WORKERS = NUM_CORES * NUM_SUBCORES
LANES = 16

B_TOTAL = 1024 * 200
B_PER_W = B_TOTAL // NUM_WORKERS
CHUNK = 128
NCHUNK = B_PER_W // CHUNK
IDX_COLS = 128
IDX_ROWS_PER_W = B_PER_W // IDX_COLS

RING = 5
AHEAD = 2


def _emb_kernel(idx_hbm, table_hbm, out_hbm, idx_v, *rest):
    bufs = rest[0:RING]
    gsems = rest[RING:2 * RING]
    wsems = rest[2 * RING:3 * RING]

    wid = lax.axis_index("s") * NUM_CORES + lax.axis_index("c")

    pltpu.sync_copy(idx_hbm.at[wid], idx_v)

    out_chunk0 = wid * NCHUNK

    def gather(g, b):
        return pltpu.make_async_copy(table_hbm.at[idx_v.at[g]], bufs[b],
                                     gsems[b])

    def write(g, b):
        row0 = (out_chunk0 + g) * CHUNK
        return pltpu.make_async_copy(bufs[b], out_hbm.at[pl.ds(row0, CHUNK)],
                                     wsems[b])

    for b in range(AHEAD):
        gather(b, b).start()

    def outer(t, carry):
        for b in range(RING):
            g = t * RING + b
            nb = (b + AHEAD) % RING

            @pl.when(g + AHEAD < NCHUNK)
            def _start_next():
                @pl.when(g >= RING - AHEAD)
                def _drain():
                    write(g, nb).wait()
                gather(g + AHEAD, nb).start()

            gather(g, b).wait()

            def row_body(i, c):
                buf = bufs[b]
                for j in range(D_MODEL // LANES):
                    sl = pl.ds(j * LANES, LANES)
                    buf[i, sl] = buf[i, sl] * SCALE
                return c

            lax.fori_loop(0, CHUNK, row_body, 0)

            write(g, b).start()
        return carry

    lax.fori_loop(0, NCHUNK // RING, outer, 0)

    for b in range(RING):
        write(0, b).wait()


@functools.partial(jax.jit, static_argnames=())
def kernel(x, table):
    idx3d = x.reshape(NUM_WORKERS, IDX_ROWS_PER_W, IDX_COLS)
    mesh = plsc.VectorSubcoreMesh(core_axis_name="c", subcore_axis_name="s")
    out = pl.kernel(
        _emb_kernel,
        mesh=mesh,
        out_type=jax.ShapeDtypeStruct((B_TOTAL, D_MODEL), jnp.float32),
        scratch_types=(
            [pltpu.VMEM((IDX_ROWS_PER_W, IDX_COLS), jnp.int32)]
            + [pltpu.VMEM((CHUNK, D_MODEL), jnp.float32) for _ in range(RING)]
            + [pltpu.SemaphoreType.DMA for _ in range(2 * RING)]
        ),
    )(idx3d, table)
    return out.reshape(x.shape[0], x.shape[1], D_MODEL)
